# initial kernel scaffold (unmeasured)
import numpy as np
import jax
import jax.numpy as jnp
from jax import lax
from jax.experimental import pallas as pl
from jax.experimental.pallas import tpu as pltpu

N_DEV = 8
B = 2
S_LOC = 512
D = 1024
HQ = 8
DH = 128
S_GLB = N_DEV * S_LOC
SCALE = 0.08838834764831843


def _rot_big() -> np.ndarray:
    r = np.zeros((DH, DH), np.float32)
    for i in range(DH // 2):
        r[2 * i + 1, 2 * i] = -1.0
        r[2 * i, 2 * i + 1] = 1.0
    big = np.zeros((D, D), np.float32)
    for h in range(HQ):
        big[h * DH:(h + 1) * DH, h * DH:(h + 1) * DH] = r
    return big


_ROT_BIG = _rot_big()


def kernel(x, Wq, Wk, Wv, Wo):
    rot_big = jnp.asarray(_ROT_BIG)

    def body(x_ref, wq_ref, wk_ref, wv_ref, wo_ref, rot_ref, out_ref,
             comm_ref, send_sems, recv_sems):
        my = lax.axis_index("i")
        left = jnp.mod(my - 1, N_DEV)
        right = jnp.mod(my + 1, N_DEV)

        barrier = pltpu.get_barrier_semaphore()
        for nbr in (left, right):
            pl.semaphore_signal(barrier, inc=1, device_id=(nbr,),
                                device_id_type=pl.DeviceIdType.MESH)
        pl.semaphore_wait(barrier, 2)

        posn = lax.broadcasted_iota(jnp.float32, (S_GLB, DH), 0)
        col = lax.broadcasted_iota(jnp.float32, (S_GLB, DH), 1)
        fidx = jnp.floor(col * 0.5)
        inv = jnp.exp(fidx * (-2.0 * np.log(10000.0) / DH))
        ang = posn * inv
        cos_t = jnp.cos(ang)
        sin_t = jnp.sin(ang)

        def rope2d(t2d, origin):
            c = lax.dynamic_slice(cos_t, (origin * S_LOC, 0), (S_LOC, DH))
            s = lax.dynamic_slice(sin_t, (origin * S_LOC, 0), (S_LOC, DH))
            c2 = jnp.concatenate([c] * HQ, axis=1)
            c2 = jnp.concatenate([c2] * B, axis=0)
            s2 = jnp.concatenate([s] * HQ, axis=1)
            s2 = jnp.concatenate([s2] * B, axis=0)
            tr = jnp.dot(t2d, rot_ref[...], preferred_element_type=jnp.float32)
            return t2d * c2 + tr * s2

        x2d = x_ref[...].reshape(B * S_LOC, D)
        q2d = rope2d(
            jnp.dot(x2d, wq_ref[...], preferred_element_type=jnp.float32), my)

        m_st = [[None] * HQ for _ in range(B)]
        l_st = [[None] * HQ for _ in range(B)]
        a_st = [[None] * HQ for _ in range(B)]

        def process(x2dj, origin):
            kj = rope2d(
                jnp.dot(x2dj, wk_ref[...], preferred_element_type=jnp.float32),
                origin)
            vj = jnp.dot(x2dj, wv_ref[...], preferred_element_type=jnp.float32)
            for b in range(B):
                rs = slice(b * S_LOC, (b + 1) * S_LOC)
                for hh in range(HQ):
                    cs = slice(hh * DH, (hh + 1) * DH)
                    q = q2d[rs, cs]
                    k = kj[rs, cs]
                    v = vj[rs, cs]
                    s = lax.dot_general(
                        q, k, (((1,), (1,)), ((), ())),
                        preferred_element_type=jnp.float32) * SCALE
                    bm = jnp.max(s, axis=1, keepdims=True)
                    if m_st[b][hh] is None:
                        m_new = bm
                        p = jnp.exp(s - m_new)
                        l_st[b][hh] = jnp.sum(p, axis=1, keepdims=True)
                        a_st[b][hh] = jnp.dot(
                            p, v, preferred_element_type=jnp.float32)
                    else:
                        m_new = jnp.maximum(m_st[b][hh], bm)
                        alpha = jnp.exp(m_st[b][hh] - m_new)
                        p = jnp.exp(s - m_new)
                        l_st[b][hh] = (l_st[b][hh] * alpha
                                       + jnp.sum(p, axis=1, keepdims=True))
                        a_st[b][hh] = (a_st[b][hh] * alpha
                                       + jnp.dot(p, v,
                                                 preferred_element_type=jnp.float32))
                    m_st[b][hh] = m_new

        for h in range(N_DEV - 1):
            src = x_ref if h == 0 else comm_ref.at[h - 1]
            rdma = pltpu.make_async_remote_copy(
                src_ref=src,
                dst_ref=comm_ref.at[h],
                send_sem=send_sems.at[h],
                recv_sem=recv_sems.at[h],
                device_id=(right,),
                device_id_type=pl.DeviceIdType.MESH,
            )
            rdma.start()
            xblk = x2d if h == 0 else comm_ref[h - 1].reshape(B * S_LOC, D)
            process(xblk, jnp.mod(my - h, N_DEV))
            rdma.wait()
        process(comm_ref[N_DEV - 2].reshape(B * S_LOC, D),
                jnp.mod(my - (N_DEV - 1), N_DEV))

        rows = []
        for b in range(B):
            rows.append(jnp.concatenate(
                [a_st[b][hh] / l_st[b][hh] for hh in range(HQ)], axis=1))
        ctx2d = jnp.concatenate(rows, axis=0)
        out2d = jnp.dot(ctx2d, wo_ref[...], preferred_element_type=jnp.float32)
        out_ref[...] = out2d.reshape(B, S_LOC, D)

    return pl.pallas_call(
        body,
        out_shape=jax.ShapeDtypeStruct((B, S_LOC, D), jnp.float32),
        in_specs=[pl.BlockSpec(memory_space=pltpu.VMEM)] * 6,
        out_specs=pl.BlockSpec(memory_space=pltpu.VMEM),
        scratch_shapes=[
            pltpu.VMEM((N_DEV - 1, B, S_LOC, D), jnp.float32),
            pltpu.SemaphoreType.DMA((N_DEV - 1,)),
            pltpu.SemaphoreType.DMA((N_DEV - 1,)),
        ],
        compiler_params=pltpu.CompilerParams(collective_id=0),
    )(x, Wq, Wk, Wv, Wo, rot_big)


# baseline (device time: 395339 ns/iter reference)
import numpy as np
import jax
import jax.numpy as jnp
from jax import lax
from jax.experimental import pallas as pl
from jax.experimental.pallas import tpu as pltpu

N_DEV = 8
B = 2
S_LOC = 512
D = 1024
HQ = 8
DH = 128
SCALE = 0.08838834764831843


def _rot() -> np.ndarray:
    r = np.zeros((DH, DH), np.float32)
    for i in range(DH // 2):
        r[2 * i + 1, 2 * i] = -1.0
        r[2 * i, 2 * i + 1] = 1.0
    return r


_ROT = _rot()


def kernel(x, Wq, Wk, Wv, Wo):
    rot = jnp.asarray(_ROT)

    def body(x_ref, wq_ref, wk_ref, wv_ref, wo_ref, rot_ref, out_ref,
             comm_ref, q_ref, acc_ref, l_ref, send_sems, recv_sems,
             credit_sem):
        my = lax.axis_index("i")
        left = jnp.mod(my - 1, N_DEV)
        right = jnp.mod(my + 1, N_DEV)

        barrier = pltpu.get_barrier_semaphore()
        for nbr in (left, right):
            pl.semaphore_signal(barrier, inc=1, device_id=(nbr,),
                                device_id_type=pl.DeviceIdType.MESH)
        pl.semaphore_wait(barrier, 2)

        row = lax.broadcasted_iota(jnp.int32, (S_LOC, DH), 0).astype(jnp.float32)
        col = lax.broadcasted_iota(jnp.int32, (S_LOC, DH), 1)
        fidx = (col // 2).astype(jnp.float32)
        inv = jnp.exp(fidx * (-2.0 * np.log(10000.0) / DH))

        def cos_sin(origin):
            ang = (row + origin.astype(jnp.float32) * S_LOC) * inv
            return jnp.cos(ang), jnp.sin(ang)

        def rope_blk(t_blk, c, s):
            tr = jnp.dot(t_blk, rot_ref[...], preferred_element_type=jnp.float32)
            return t_blk * c + tr * s

        my_c, my_s = cos_sin(my)

        def q_init(idx, carry):
            b, hh = idx // HQ, idx % HQ
            xb = x_ref[pl.ds(b, 1), :, :].reshape(S_LOC, D)
            qb = jnp.dot(xb, wq_ref[:, pl.ds(hh * DH, DH)],
                         preferred_element_type=jnp.float32)
            q_ref[pl.ds(b * S_LOC, S_LOC), pl.ds(hh * DH, DH)] = (
                rope_blk(qb, my_c, my_s))
            return carry

        lax.fori_loop(0, B * HQ, q_init, 0)

        def process(hop, get_rows, origin):
            c, s = cos_sin(origin)

            def blk(idx, carry):
                b, hh = idx // HQ, idx % HQ
                xb = get_rows(b)
                rsq = pl.ds(b * S_LOC, S_LOC)
                csq = pl.ds(hh * DH, DH)
                k = rope_blk(
                    jnp.dot(xb, wk_ref[:, csq],
                            preferred_element_type=jnp.float32), c, s)
                v = jnp.dot(xb, wv_ref[:, csq],
                            preferred_element_type=jnp.float32)
                sc = lax.dot_general(
                    q_ref[rsq, csq], k, (((1,), (1,)), ((), ())),
                    preferred_element_type=jnp.float32) * SCALE
                p = jnp.exp(sc)
                lb = jnp.sum(p, axis=1, keepdims=True).reshape(1, S_LOC, 1)
                pv = jnp.dot(p, v, preferred_element_type=jnp.float32)
                if hop == 0:
                    l_ref[pl.ds(idx, 1)] = lb
                    acc_ref[rsq, csq] = pv
                else:
                    l_ref[pl.ds(idx, 1)] = l_ref[pl.ds(idx, 1)] + lb
                    acc_ref[rsq, csq] = acc_ref[rsq, csq] + pv
                return carry

            lax.fori_loop(0, B * HQ, blk, 0)

        for h in range(N_DEV - 1):
            if h >= 2:
                pl.semaphore_wait(credit_sem, 1)
            src = x_ref if h == 0 else comm_ref.at[(h - 1) % 2]
            rdma = pltpu.make_async_remote_copy(
                src_ref=src,
                dst_ref=comm_ref.at[h % 2],
                send_sem=send_sems.at[h % 2],
                recv_sem=recv_sems.at[h % 2],
                device_id=(right,),
                device_id_type=pl.DeviceIdType.MESH,
            )
            rdma.start()
            if h == 0:
                get_rows = lambda b: x_ref[pl.ds(b, 1), :, :].reshape(S_LOC, D)
            else:
                slot = (h - 1) % 2
                get_rows = lambda b, _s=slot: comm_ref[
                    _s, pl.ds(b, 1), :, :].reshape(S_LOC, D)
            process(h, get_rows, jnp.mod(my - h, N_DEV))
            rdma.wait()
            if 1 <= h <= N_DEV - 3:
                pl.semaphore_signal(credit_sem, inc=1, device_id=(left,),
                                    device_id_type=pl.DeviceIdType.MESH)
        process(N_DEV - 1,
                lambda b: comm_ref[(N_DEV - 2) % 2,
                                   pl.ds(b, 1), :, :].reshape(S_LOC, D),
                jnp.mod(my + 1, N_DEV))

        def norm(idx, carry):
            b, hh = idx // HQ, idx % HQ
            rsq = pl.ds(b * S_LOC, S_LOC)
            csq = pl.ds(hh * DH, DH)
            lb = l_ref[pl.ds(idx, 1)].reshape(S_LOC, 1)
            acc_ref[rsq, csq] = acc_ref[rsq, csq] / lb
            return carry

        lax.fori_loop(0, B * HQ, norm, 0)
        out2d = jnp.dot(acc_ref[...], wo_ref[...],
                        preferred_element_type=jnp.float32)
        out_ref[...] = out2d.reshape(B, S_LOC, D)

    return pl.pallas_call(
        body,
        out_shape=jax.ShapeDtypeStruct((B, S_LOC, D), jnp.float32),
        in_specs=[pl.BlockSpec(memory_space=pltpu.VMEM)] * 6,
        out_specs=pl.BlockSpec(memory_space=pltpu.VMEM),
        scratch_shapes=[
            pltpu.VMEM((2, B, S_LOC, D), jnp.float32),
            pltpu.VMEM((B * S_LOC, D), jnp.float32),
            pltpu.VMEM((B * S_LOC, D), jnp.float32),
            pltpu.VMEM((B * HQ, S_LOC, 1), jnp.float32),
            pltpu.SemaphoreType.DMA((2,)),
            pltpu.SemaphoreType.DMA((2,)),
            pltpu.SemaphoreType.REGULAR,
        ],
        compiler_params=pltpu.CompilerParams(
            collective_id=0, vmem_limit_bytes=34 * 1024 * 1024),
    )(x, Wq, Wk, Wv, Wo, rot)


# device time: 256959 ns/iter; 1.5385x vs baseline; 1.5385x over previous
import numpy as np
import jax
import jax.numpy as jnp
from jax import lax
from jax.experimental import pallas as pl
from jax.experimental.pallas import tpu as pltpu

N_DEV = 8
B = 2
S_LOC = 512
D = 1024
HQ = 8
DH = 128
SCALE = 0.08838834764831843


def _rot() -> np.ndarray:
    r = np.zeros((DH, DH), np.float32)
    for i in range(DH // 2):
        r[2 * i + 1, 2 * i] = -1.0
        r[2 * i, 2 * i + 1] = 1.0
    return r


_ROT = _rot()


def kernel(x, Wq, Wk, Wv, Wo):
    rot = jnp.asarray(_ROT)

    def body(x_ref, wq_ref, wk_ref, wv_ref, wo_ref, rot_ref, out_ref,
             comm_ref, x16_ref, q_ref, acc_ref, l_ref, send_sems, recv_sems,
             credit_sem):
        my = lax.axis_index("i")
        left = jnp.mod(my - 1, N_DEV)
        right = jnp.mod(my + 1, N_DEV)

        barrier = pltpu.get_barrier_semaphore()
        for nbr in (left, right):
            pl.semaphore_signal(barrier, inc=1, device_id=(nbr,),
                                device_id_type=pl.DeviceIdType.MESH)
        pl.semaphore_wait(barrier, 2)

        row = lax.broadcasted_iota(jnp.int32, (S_LOC, DH), 0).astype(jnp.float32)
        col = lax.broadcasted_iota(jnp.int32, (S_LOC, DH), 1)
        fidx = (col // 2).astype(jnp.float32)
        inv = jnp.exp(fidx * (-2.0 * np.log(10000.0) / DH))

        def cos_sin(origin):
            ang = (row + origin.astype(jnp.float32) * S_LOC) * inv
            return jnp.cos(ang), jnp.sin(ang)

        def rope_blk(t_blk, c, s):
            tr = jnp.dot(t_blk, rot_ref[...], preferred_element_type=jnp.float32)
            return t_blk * c + tr * s

        x16_ref[...] = x_ref[...].astype(jnp.bfloat16)

        my_c, my_s = cos_sin(my)

        def q_init(idx, carry):
            b, hh = idx // HQ, idx % HQ
            xb = x16_ref[pl.ds(b, 1), :, :].reshape(S_LOC, D)
            qb = jnp.dot(xb, wq_ref[:, pl.ds(hh * DH, DH)].astype(jnp.bfloat16),
                         preferred_element_type=jnp.float32)
            q_ref[pl.ds(b * S_LOC, S_LOC), pl.ds(hh * DH, DH)] = (
                rope_blk(qb, my_c, my_s).astype(jnp.bfloat16))
            return carry

        lax.fori_loop(0, B * HQ, q_init, 0)

        def process(hop, get_rows, origin):
            c, s = cos_sin(origin)

            def blk(idx, carry):
                b, hh = idx // HQ, idx % HQ
                xb = get_rows(b)
                rsq = pl.ds(b * S_LOC, S_LOC)
                csq = pl.ds(hh * DH, DH)
                k = rope_blk(
                    jnp.dot(xb, wk_ref[:, csq].astype(jnp.bfloat16),
                            preferred_element_type=jnp.float32), c, s)
                v = jnp.dot(xb, wv_ref[:, csq].astype(jnp.bfloat16),
                            preferred_element_type=jnp.float32)
                sc = lax.dot_general(
                    q_ref[rsq, csq], k.astype(jnp.bfloat16),
                    (((1,), (1,)), ((), ())),
                    preferred_element_type=jnp.float32) * SCALE
                p = jnp.exp(sc)
                lb = jnp.sum(p, axis=1, keepdims=True).reshape(1, S_LOC, 1)
                pv = jnp.dot(p.astype(jnp.bfloat16), v.astype(jnp.bfloat16),
                             preferred_element_type=jnp.float32)
                if hop == 0:
                    l_ref[pl.ds(idx, 1)] = lb
                    acc_ref[rsq, csq] = pv
                else:
                    l_ref[pl.ds(idx, 1)] = l_ref[pl.ds(idx, 1)] + lb
                    acc_ref[rsq, csq] = acc_ref[rsq, csq] + pv
                return carry

            lax.fori_loop(0, B * HQ, blk, 0)

        for h in range(N_DEV - 1):
            if h >= 2:
                pl.semaphore_wait(credit_sem, 1)
            src = x16_ref if h == 0 else comm_ref.at[(h - 1) % 2]
            rdma = pltpu.make_async_remote_copy(
                src_ref=src,
                dst_ref=comm_ref.at[h % 2],
                send_sem=send_sems.at[h % 2],
                recv_sem=recv_sems.at[h % 2],
                device_id=(right,),
                device_id_type=pl.DeviceIdType.MESH,
            )
            rdma.start()
            if h == 0:
                get_rows = lambda b: x16_ref[pl.ds(b, 1), :, :].reshape(S_LOC, D)
            else:
                slot = (h - 1) % 2
                get_rows = lambda b, _s=slot: comm_ref[
                    _s, pl.ds(b, 1), :, :].reshape(S_LOC, D)
            process(h, get_rows, jnp.mod(my - h, N_DEV))
            rdma.wait()
            if 1 <= h <= N_DEV - 3:
                pl.semaphore_signal(credit_sem, inc=1, device_id=(left,),
                                    device_id_type=pl.DeviceIdType.MESH)
        process(N_DEV - 1,
                lambda b: comm_ref[(N_DEV - 2) % 2,
                                   pl.ds(b, 1), :, :].reshape(S_LOC, D),
                jnp.mod(my + 1, N_DEV))

        def norm(idx, carry):
            b, hh = idx // HQ, idx % HQ
            rsq = pl.ds(b * S_LOC, S_LOC)
            csq = pl.ds(hh * DH, DH)
            lb = l_ref[pl.ds(idx, 1)].reshape(S_LOC, 1)
            acc_ref[rsq, csq] = acc_ref[rsq, csq] / lb
            return carry

        lax.fori_loop(0, B * HQ, norm, 0)
        out2d = jnp.dot(acc_ref[...].astype(jnp.bfloat16),
                        wo_ref[...].astype(jnp.bfloat16),
                        preferred_element_type=jnp.float32)
        out_ref[...] = out2d.reshape(B, S_LOC, D)

    return pl.pallas_call(
        body,
        out_shape=jax.ShapeDtypeStruct((B, S_LOC, D), jnp.float32),
        in_specs=[pl.BlockSpec(memory_space=pltpu.VMEM)] * 6,
        out_specs=pl.BlockSpec(memory_space=pltpu.VMEM),
        scratch_shapes=[
            pltpu.VMEM((2, B, S_LOC, D), jnp.bfloat16),
            pltpu.VMEM((B, S_LOC, D), jnp.bfloat16),
            pltpu.VMEM((B * S_LOC, D), jnp.bfloat16),
            pltpu.VMEM((B * S_LOC, D), jnp.float32),
            pltpu.VMEM((B * HQ, S_LOC, 1), jnp.float32),
            pltpu.SemaphoreType.DMA((2,)),
            pltpu.SemaphoreType.DMA((2,)),
            pltpu.SemaphoreType.REGULAR,
        ],
        compiler_params=pltpu.CompilerParams(
            collective_id=0, vmem_limit_bytes=34 * 1024 * 1024),
    )(x, Wq, Wk, Wv, Wo, rot)


# device time: 220089 ns/iter; 1.7963x vs baseline; 1.1675x over previous
import numpy as np
import jax
import jax.numpy as jnp
from jax import lax
from jax.experimental import pallas as pl
from jax.experimental.pallas import tpu as pltpu

N_DEV = 8
B = 2
S_LOC = 512
D = 1024
HQ = 8
DH = 128
SCALE = 0.08838834764831843
BF = jnp.bfloat16


def _rot_big() -> np.ndarray:
    r = np.zeros((DH, DH), np.float32)
    for i in range(DH // 2):
        r[2 * i + 1, 2 * i] = -1.0
        r[2 * i, 2 * i + 1] = 1.0
    big = np.zeros((D, D), np.float32)
    for h in range(HQ):
        big[h * DH:(h + 1) * DH, h * DH:(h + 1) * DH] = r
    return big


_ROT_BIG = _rot_big()


def kernel(x, Wq, Wk, Wv, Wo):
    rot_big = jnp.asarray(_ROT_BIG, dtype=BF)

    def body(x_ref, wq_ref, wk_ref, wv_ref, wo_ref, rot_ref, out_ref,
             comm_ref, x16_ref, q_ref, k_ref, v_ref, acc_ref, l_ref,
             send_sems, recv_sems, credit_sem):
        my = lax.axis_index("i")
        left = jnp.mod(my - 1, N_DEV)
        right = jnp.mod(my + 1, N_DEV)

        barrier = pltpu.get_barrier_semaphore()
        for nbr in (left, right):
            pl.semaphore_signal(barrier, inc=1, device_id=(nbr,),
                                device_id_type=pl.DeviceIdType.MESH)
        pl.semaphore_wait(barrier, 2)

        row = lax.broadcasted_iota(jnp.int32, (S_LOC, DH), 0).astype(jnp.float32)
        col = lax.broadcasted_iota(jnp.int32, (S_LOC, DH), 1)
        fidx = (col // 2).astype(jnp.float32)
        inv = jnp.exp(fidx * (-2.0 * np.log(10000.0) / DH))

        def cos_sin_big(origin):
            ang = (row + origin.astype(jnp.float32) * S_LOC) * inv
            c = jnp.cos(ang).astype(BF)
            s = jnp.sin(ang).astype(BF)
            c = jnp.concatenate([c] * HQ, axis=1)
            s = jnp.concatenate([s] * HQ, axis=1)
            return (jnp.concatenate([c] * B, axis=0),
                    jnp.concatenate([s] * B, axis=0))

        def rope_full(tf32, cbig, sbig):
            t16 = tf32.astype(BF)
            tr = jnp.dot(t16, rot_ref[...],
                         preferred_element_type=jnp.float32).astype(BF)
            return t16 * cbig + tr * sbig

        x16_ref[...] = x_ref[...].astype(BF)

        my_c, my_s = cos_sin_big(my)
        x16full = x16_ref[...].reshape(B * S_LOC, D)
        q_ref[...] = rope_full(
            jnp.dot(x16full, wq_ref[...].astype(BF),
                    preferred_element_type=jnp.float32), my_c, my_s)

        def process(hop, xfull16, origin):
            cbig, sbig = cos_sin_big(origin)
            k_ref[...] = rope_full(
                jnp.dot(xfull16, wk_ref[...].astype(BF),
                        preferred_element_type=jnp.float32), cbig, sbig)
            v_ref[...] = jnp.dot(xfull16, wv_ref[...].astype(BF),
                                 preferred_element_type=jnp.float32).astype(BF)

            def blk(idx, carry):
                b, hh = idx // HQ, idx % HQ
                rsq = pl.ds(b * S_LOC, S_LOC)
                csq = pl.ds(hh * DH, DH)
                sc = lax.dot_general(
                    q_ref[rsq, csq], k_ref[rsq, csq],
                    (((1,), (1,)), ((), ())),
                    preferred_element_type=jnp.float32) * SCALE
                p = jnp.exp(sc)
                lb = jnp.sum(p, axis=1, keepdims=True).reshape(1, S_LOC, 1)
                pv = jnp.dot(p.astype(BF), v_ref[rsq, csq],
                             preferred_element_type=jnp.float32)
                if hop == 0:
                    l_ref[pl.ds(idx, 1)] = lb
                    acc_ref[rsq, csq] = pv
                else:
                    l_ref[pl.ds(idx, 1)] = l_ref[pl.ds(idx, 1)] + lb
                    acc_ref[rsq, csq] = acc_ref[rsq, csq] + pv
                return carry

            lax.fori_loop(0, B * HQ, blk, 0)

        for h in range(N_DEV - 1):
            if h >= 2:
                pl.semaphore_wait(credit_sem, 1)
            src = x16_ref if h == 0 else comm_ref.at[(h - 1) % 2]
            rdma = pltpu.make_async_remote_copy(
                src_ref=src,
                dst_ref=comm_ref.at[h % 2],
                send_sem=send_sems.at[h % 2],
                recv_sem=recv_sems.at[h % 2],
                device_id=(right,),
                device_id_type=pl.DeviceIdType.MESH,
            )
            rdma.start()
            if h == 0:
                xfull16 = x16full
            else:
                xfull16 = comm_ref[(h - 1) % 2].reshape(B * S_LOC, D)
            process(h, xfull16, jnp.mod(my - h, N_DEV))
            rdma.wait()
            if 1 <= h <= N_DEV - 3:
                pl.semaphore_signal(credit_sem, inc=1, device_id=(left,),
                                    device_id_type=pl.DeviceIdType.MESH)
        process(N_DEV - 1, comm_ref[(N_DEV - 2) % 2].reshape(B * S_LOC, D),
                jnp.mod(my + 1, N_DEV))

        def norm(idx, carry):
            b, hh = idx // HQ, idx % HQ
            rsq = pl.ds(b * S_LOC, S_LOC)
            csq = pl.ds(hh * DH, DH)
            lb = l_ref[pl.ds(idx, 1)].reshape(S_LOC, 1)
            acc_ref[rsq, csq] = acc_ref[rsq, csq] / lb
            return carry

        lax.fori_loop(0, B * HQ, norm, 0)
        out2d = jnp.dot(acc_ref[...].astype(BF), wo_ref[...].astype(BF),
                        preferred_element_type=jnp.float32)
        out_ref[...] = out2d.reshape(B, S_LOC, D)

    return pl.pallas_call(
        body,
        out_shape=jax.ShapeDtypeStruct((B, S_LOC, D), jnp.float32),
        in_specs=[pl.BlockSpec(memory_space=pltpu.VMEM)] * 6,
        out_specs=pl.BlockSpec(memory_space=pltpu.VMEM),
        scratch_shapes=[
            pltpu.VMEM((2, B, S_LOC, D), BF),
            pltpu.VMEM((B, S_LOC, D), BF),
            pltpu.VMEM((B * S_LOC, D), BF),
            pltpu.VMEM((B * S_LOC, D), BF),
            pltpu.VMEM((B * S_LOC, D), BF),
            pltpu.VMEM((B * S_LOC, D), jnp.float32),
            pltpu.VMEM((B * HQ, S_LOC, 1), jnp.float32),
            pltpu.SemaphoreType.DMA((2,)),
            pltpu.SemaphoreType.DMA((2,)),
            pltpu.SemaphoreType.REGULAR,
        ],
        compiler_params=pltpu.CompilerParams(
            collective_id=0, vmem_limit_bytes=35 * 1024 * 1024),
    )(x, Wq, Wk, Wv, Wo, rot_big)


# device time: 207668 ns/iter; 1.9037x vs baseline; 1.0598x over previous
import numpy as np
import jax
import jax.numpy as jnp
from jax import lax
from jax.experimental import pallas as pl
from jax.experimental.pallas import tpu as pltpu

N_DEV = 8
B = 2
S_LOC = 512
D = 1024
HQ = 8
DH = 128
SCALE = 0.08838834764831843
BF = jnp.bfloat16


def _rot_big() -> np.ndarray:
    r = np.zeros((DH, DH), np.float32)
    for i in range(DH // 2):
        r[2 * i + 1, 2 * i] = -1.0
        r[2 * i, 2 * i + 1] = 1.0
    big = np.zeros((D, D), np.float32)
    for h in range(HQ):
        big[h * DH:(h + 1) * DH, h * DH:(h + 1) * DH] = r
    return big


_ROT_BIG = _rot_big()


def kernel(x, Wq, Wk, Wv, Wo):
    rot_big = jnp.asarray(_ROT_BIG, dtype=BF)

    def body(x_ref, wq_ref, wk_ref, wv_ref, wo_ref, rot_ref, out_ref,
             comm_ref, q_ref, k_ref, v_ref, acc_ref, l_ref,
             send_sems, recv_sems, credit_sem):
        my = lax.axis_index("i")
        left = jnp.mod(my - 1, N_DEV)
        right = jnp.mod(my + 1, N_DEV)

        barrier = pltpu.get_barrier_semaphore()
        for nbr in (left, right):
            pl.semaphore_signal(barrier, inc=1, device_id=(nbr,),
                                device_id_type=pl.DeviceIdType.MESH)
        pl.semaphore_wait(barrier, 2)

        row = lax.broadcasted_iota(jnp.int32, (S_LOC, DH), 0).astype(jnp.float32)
        col = lax.broadcasted_iota(jnp.int32, (S_LOC, DH), 1)
        fidx = (col // 2).astype(jnp.float32)
        inv = jnp.exp(fidx * (-2.0 * np.log(10000.0) / DH))

        def cos_sin_big(origin):
            ang = (row + origin.astype(jnp.float32) * S_LOC) * inv
            c = jnp.cos(ang).astype(BF)
            s = jnp.sin(ang).astype(BF)
            c = jnp.concatenate([c] * HQ, axis=1)
            s = jnp.concatenate([s] * HQ, axis=1)
            return (jnp.concatenate([c] * B, axis=0),
                    jnp.concatenate([s] * B, axis=0))

        def rope_full(tf32, cbig, sbig):
            t16 = tf32.astype(BF)
            tr = jnp.dot(t16, rot_ref[...],
                         preferred_element_type=jnp.float32).astype(BF)
            return t16 * cbig + tr * sbig

        comm_ref[2] = x_ref[...].astype(BF)
        x16full = comm_ref[2].reshape(B * S_LOC, D)

        def process(hop, xfull16, origin):
            cbig, sbig = cos_sin_big(origin)
            k_ref[...] = rope_full(
                jnp.dot(xfull16, wk_ref[...].astype(BF),
                        preferred_element_type=jnp.float32), cbig, sbig)
            v_ref[...] = jnp.dot(xfull16, wv_ref[...].astype(BF),
                                 preferred_element_type=jnp.float32).astype(BF)

            def blk(idx, carry):
                b, hh = idx // HQ, idx % HQ
                rsq = pl.ds(b * S_LOC, S_LOC)
                csq = pl.ds(hh * DH, DH)
                sc = lax.dot_general(
                    q_ref[rsq, csq], k_ref[rsq, csq],
                    (((1,), (1,)), ((), ())),
                    preferred_element_type=jnp.float32) * SCALE
                p = jnp.exp(sc)
                lb = jnp.sum(p, axis=1, keepdims=True).reshape(1, S_LOC, 1)
                pv = jnp.dot(p.astype(BF), v_ref[rsq, csq],
                             preferred_element_type=jnp.float32)
                if hop == 0:
                    l_ref[pl.ds(idx, 1)] = lb
                    acc_ref[rsq, csq] = pv
                else:
                    l_ref[pl.ds(idx, 1)] = l_ref[pl.ds(idx, 1)] + lb
                    acc_ref[rsq, csq] = acc_ref[rsq, csq] + pv
                return carry

            lax.fori_loop(0, B * HQ, blk, 0)

        for h in range(N_DEV - 1):
            if h >= 2:
                pl.semaphore_wait(credit_sem, 1)
            src = comm_ref.at[2] if h == 0 else comm_ref.at[(h - 1) % 3]
            rdma = pltpu.make_async_remote_copy(
                src_ref=src,
                dst_ref=comm_ref.at[h % 3],
                send_sem=send_sems.at[h],
                recv_sem=recv_sems.at[h],
                device_id=(right,),
                device_id_type=pl.DeviceIdType.MESH,
            )
            rdma.start()
            if h == 0:
                my_c, my_s = cos_sin_big(my)
                q_ref[...] = rope_full(
                    jnp.dot(x16full, wq_ref[...].astype(BF),
                            preferred_element_type=jnp.float32), my_c, my_s)
                xfull16 = x16full
            else:
                xfull16 = comm_ref[(h - 1) % 3].reshape(B * S_LOC, D)
            process(h, xfull16, jnp.mod(my - h, N_DEV))
            rdma.wait()
            if h <= N_DEV - 4:
                pl.semaphore_signal(credit_sem, inc=1, device_id=(left,),
                                    device_id_type=pl.DeviceIdType.MESH)
        process(N_DEV - 1, comm_ref[(N_DEV - 2) % 3].reshape(B * S_LOC, D),
                jnp.mod(my + 1, N_DEV))

        def norm(idx, carry):
            b, hh = idx // HQ, idx % HQ
            rsq = pl.ds(b * S_LOC, S_LOC)
            csq = pl.ds(hh * DH, DH)
            lb = l_ref[pl.ds(idx, 1)].reshape(S_LOC, 1)
            acc_ref[rsq, csq] = acc_ref[rsq, csq] / lb
            return carry

        lax.fori_loop(0, B * HQ, norm, 0)
        out2d = jnp.dot(acc_ref[...].astype(BF), wo_ref[...].astype(BF),
                        preferred_element_type=jnp.float32)
        out_ref[...] = out2d.reshape(B, S_LOC, D)

    return pl.pallas_call(
        body,
        out_shape=jax.ShapeDtypeStruct((B, S_LOC, D), jnp.float32),
        in_specs=[pl.BlockSpec(memory_space=pltpu.VMEM)] * 6,
        out_specs=pl.BlockSpec(memory_space=pltpu.VMEM),
        scratch_shapes=[
            pltpu.VMEM((3, B, S_LOC, D), BF),
            pltpu.VMEM((B * S_LOC, D), BF),
            pltpu.VMEM((B * S_LOC, D), BF),
            pltpu.VMEM((B * S_LOC, D), BF),
            pltpu.VMEM((B * S_LOC, D), jnp.float32),
            pltpu.VMEM((B * HQ, S_LOC, 1), jnp.float32),
            pltpu.SemaphoreType.DMA((N_DEV - 1,)),
            pltpu.SemaphoreType.DMA((N_DEV - 1,)),
            pltpu.SemaphoreType.REGULAR,
        ],
        compiler_params=pltpu.CompilerParams(
            collective_id=0, vmem_limit_bytes=35 * 1024 * 1024),
    )(x, Wq, Wk, Wv, Wo, rot_big)


# device time: 203306 ns/iter; 1.9446x vs baseline; 1.0215x over previous
import numpy as np
import jax
import jax.numpy as jnp
from jax import lax
from jax.experimental import pallas as pl
from jax.experimental.pallas import tpu as pltpu

N_DEV = 8
B = 2
S_LOC = 512
D = 1024
HQ = 8
DH = 128
SCALE = 0.08838834764831843
BF = jnp.bfloat16


def _rot_big() -> np.ndarray:
    r = np.zeros((DH, DH), np.float32)
    for i in range(DH // 2):
        r[2 * i + 1, 2 * i] = -1.0
        r[2 * i, 2 * i + 1] = 1.0
    big = np.zeros((D, D), np.float32)
    for h in range(HQ):
        big[h * DH:(h + 1) * DH, h * DH:(h + 1) * DH] = r
    return big


_ROT_BIG = _rot_big()


def kernel(x, Wq, Wk, Wv, Wo):
    def body(x_ref, wq_ref, wk_ref, wv_ref, wo_ref, out_ref,
             comm_ref, q_ref, k_ref, v_ref, acc_ref, l_ref,
             send_sems, recv_sems, credit_sem):
        my = lax.axis_index("i")
        left = jnp.mod(my - 1, N_DEV)
        right = jnp.mod(my + 1, N_DEV)

        barrier = pltpu.get_barrier_semaphore()
        for nbr in (left, right):
            pl.semaphore_signal(barrier, inc=1, device_id=(nbr,),
                                device_id_type=pl.DeviceIdType.MESH)
        pl.semaphore_wait(barrier, 2)

        row = lax.broadcasted_iota(jnp.int32, (S_LOC, DH), 0).astype(jnp.float32)
        col = lax.broadcasted_iota(jnp.int32, (S_LOC, DH), 1)
        fidx = (col // 2).astype(jnp.float32)
        inv = jnp.exp(fidx * (-2.0 * np.log(10000.0) / DH))

        def cos_sin_big(origin):
            ang = (row + origin.astype(jnp.float32) * S_LOC) * inv
            c = jnp.cos(ang).astype(BF)
            s = jnp.sin(ang).astype(BF)
            c = jnp.concatenate([c] * HQ, axis=1)
            s = jnp.concatenate([s] * HQ, axis=1)
            return (jnp.concatenate([c] * B, axis=0),
                    jnp.concatenate([s] * B, axis=0))

        lane = lax.broadcasted_iota(jnp.int32, (B * S_LOC, D), 1)
        even_lane = (lane % 2) == 0

        def rope_full(tf32, cbig, sbig):
            t16 = tf32.astype(BF)
            tr = jnp.where(even_lane,
                           -pltpu.roll(t16, D - 1, axis=1),
                           pltpu.roll(t16, 1, axis=1))
            return t16 * cbig + tr * sbig

        comm_ref[2] = x_ref[...].astype(BF)
        x16full = comm_ref[2].reshape(B * S_LOC, D)

        def process(hop, xfull16, origin):
            cbig, sbig = cos_sin_big(origin)
            k_ref[...] = rope_full(
                jnp.dot(xfull16, wk_ref[...].astype(BF),
                        preferred_element_type=jnp.float32), cbig, sbig)
            v_ref[...] = jnp.dot(xfull16, wv_ref[...].astype(BF),
                                 preferred_element_type=jnp.float32).astype(BF)

            def blk(idx, carry):
                b, hh = idx // HQ, idx % HQ
                rsq = pl.ds(b * S_LOC, S_LOC)
                csq = pl.ds(hh * DH, DH)
                sc = lax.dot_general(
                    q_ref[rsq, csq], k_ref[rsq, csq],
                    (((1,), (1,)), ((), ())),
                    preferred_element_type=jnp.float32) * SCALE
                p = jnp.exp(sc)
                lb = jnp.sum(p, axis=1, keepdims=True).reshape(1, S_LOC, 1)
                pv = jnp.dot(p.astype(BF), v_ref[rsq, csq],
                             preferred_element_type=jnp.float32)
                if hop == 0:
                    l_ref[pl.ds(idx, 1)] = lb
                    acc_ref[rsq, csq] = pv
                else:
                    l_ref[pl.ds(idx, 1)] = l_ref[pl.ds(idx, 1)] + lb
                    acc_ref[rsq, csq] = acc_ref[rsq, csq] + pv
                return carry

            lax.fori_loop(0, B * HQ, blk, 0)

        for h in range(N_DEV - 1):
            if h >= 2:
                pl.semaphore_wait(credit_sem, 1)
            src = comm_ref.at[2] if h == 0 else comm_ref.at[(h - 1) % 3]
            rdma = pltpu.make_async_remote_copy(
                src_ref=src,
                dst_ref=comm_ref.at[h % 3],
                send_sem=send_sems.at[h],
                recv_sem=recv_sems.at[h],
                device_id=(right,),
                device_id_type=pl.DeviceIdType.MESH,
            )
            rdma.start()
            if h == 0:
                my_c, my_s = cos_sin_big(my)
                q_ref[...] = rope_full(
                    jnp.dot(x16full, wq_ref[...].astype(BF),
                            preferred_element_type=jnp.float32), my_c, my_s)
                xfull16 = x16full
            else:
                xfull16 = comm_ref[(h - 1) % 3].reshape(B * S_LOC, D)
            process(h, xfull16, jnp.mod(my - h, N_DEV))
            rdma.wait()
            if h <= N_DEV - 4:
                pl.semaphore_signal(credit_sem, inc=1, device_id=(left,),
                                    device_id_type=pl.DeviceIdType.MESH)
        process(N_DEV - 1, comm_ref[(N_DEV - 2) % 3].reshape(B * S_LOC, D),
                jnp.mod(my + 1, N_DEV))

        def norm(idx, carry):
            b, hh = idx // HQ, idx % HQ
            rsq = pl.ds(b * S_LOC, S_LOC)
            csq = pl.ds(hh * DH, DH)
            lb = l_ref[pl.ds(idx, 1)].reshape(S_LOC, 1)
            acc_ref[rsq, csq] = acc_ref[rsq, csq] / lb
            return carry

        lax.fori_loop(0, B * HQ, norm, 0)
        out2d = jnp.dot(acc_ref[...].astype(BF), wo_ref[...].astype(BF),
                        preferred_element_type=jnp.float32)
        out_ref[...] = out2d.reshape(B, S_LOC, D)

    return pl.pallas_call(
        body,
        out_shape=jax.ShapeDtypeStruct((B, S_LOC, D), jnp.float32),
        in_specs=[pl.BlockSpec(memory_space=pltpu.VMEM)] * 5,
        out_specs=pl.BlockSpec(memory_space=pltpu.VMEM),
        scratch_shapes=[
            pltpu.VMEM((3, B, S_LOC, D), BF),
            pltpu.VMEM((B * S_LOC, D), BF),
            pltpu.VMEM((B * S_LOC, D), BF),
            pltpu.VMEM((B * S_LOC, D), BF),
            pltpu.VMEM((B * S_LOC, D), jnp.float32),
            pltpu.VMEM((B * HQ, S_LOC, 1), jnp.float32),
            pltpu.SemaphoreType.DMA((N_DEV - 1,)),
            pltpu.SemaphoreType.DMA((N_DEV - 1,)),
            pltpu.SemaphoreType.REGULAR,
        ],
        compiler_params=pltpu.CompilerParams(
            collective_id=0, vmem_limit_bytes=35 * 1024 * 1024),
    )(x, Wq, Wk, Wv, Wo)
